# baseline (speedup 1.0000x reference)
import math

import jax
import jax.numpy as jnp
from jax.experimental import pallas as pl

N = 20000
C = 91
SCORE_THRESH = 0.05
NMS_THRESH = 0.5
DET_PER_IMG = 300
TOPK = 1000
IMG_H = 800
IMG_W = 1066
BBOX_XFORM_CLIP = math.log(1000.0 / 16.0)
NEG = -1e9

PAD = 1024


def _nms_kernel(sc_ref, lb_ref, dx_ref, dy_ref, dw_ref, dh_ref,
                a1_ref, a2_ref, a3_ref, a4_ref,
                bx1_ref, by1_ref, bx2_ref, by2_ref, os_ref, ol_ref):
    score = sc_ref[...]
    label = lb_ref[...]
    a1 = a1_ref[...]
    a2 = a2_ref[...]
    a3 = a3_ref[...]
    a4 = a4_ref[...]

    widths = a3 - a1
    heights = a4 - a2
    ctr_x = a1 + 0.5 * widths
    ctr_y = a2 + 0.5 * heights
    dw = jnp.minimum(dw_ref[...], BBOX_XFORM_CLIP)
    dh = jnp.minimum(dh_ref[...], BBOX_XFORM_CLIP)
    pred_ctr_x = dx_ref[...] * widths + ctr_x
    pred_ctr_y = dy_ref[...] * heights + ctr_y
    pred_w = jnp.exp(dw) * widths
    pred_h = jnp.exp(dh) * heights
    x1 = jnp.clip(pred_ctr_x - 0.5 * pred_w, 0.0, float(IMG_W))
    y1 = jnp.clip(pred_ctr_y - 0.5 * pred_h, 0.0, float(IMG_H))
    x2 = jnp.clip(pred_ctr_x + 0.5 * pred_w, 0.0, float(IMG_W))
    y2 = jnp.clip(pred_ctr_y + 0.5 * pred_h, 0.0, float(IMG_H))

    maxbox = jnp.max(jnp.maximum(jnp.maximum(x1, y1), jnp.maximum(x2, y2)))
    off = label.astype(jnp.float32) * (maxbox + 1.0)
    o1 = x1 + off
    o2 = y1 + off
    o3 = x2 + off
    o4 = y2 + off
    area2 = (o3 - o1) * (o4 - o2)

    iota = (jax.lax.broadcasted_iota(jnp.int32, (8, 128), 0) * 128
            + jax.lax.broadcasted_iota(jnp.int32, (8, 128), 1))
    s0 = jnp.where(score > SCORE_THRESH, score, NEG)

    zf = jnp.zeros((8, 128), jnp.float32)
    init = (s0, zf, zf, zf, zf, zf, jnp.full((8, 128), -1, jnp.int32))

    def body(i, carry):
        s, b1, b2, b3, b4, bs, bl = carry
        m = jnp.max(s)
        eq = s == m
        idx = jnp.min(jnp.where(eq, iota, jnp.int32(1 << 30)))
        valid = m > NEG / 2.0
        ch = iota == idx
        rb1 = jnp.sum(jnp.where(ch, x1, 0.0))
        rb2 = jnp.sum(jnp.where(ch, y1, 0.0))
        rb3 = jnp.sum(jnp.where(ch, x2, 0.0))
        rb4 = jnp.sum(jnp.where(ch, y2, 0.0))
        coff = jnp.sum(jnp.where(ch, off, 0.0))
        csc = jnp.sum(jnp.where(ch, score, 0.0))
        clab = jnp.sum(jnp.where(ch, label, 0))
        cx1 = rb1 + coff
        cy1 = rb2 + coff
        cx2 = rb3 + coff
        cy2 = rb4 + coff
        area1 = (cx2 - cx1) * (cy2 - cy1)
        ltx = jnp.maximum(cx1, o1)
        lty = jnp.maximum(cy1, o2)
        rbx = jnp.minimum(cx2, o3)
        rby = jnp.minimum(cy2, o4)
        w = jnp.maximum(rbx - ltx, 0.0)
        h = jnp.maximum(rby - lty, 0.0)
        inter = w * h
        iou = inter / (area1 + area2 - inter + 1e-12)
        s = jnp.where(iou > NMS_THRESH, NEG, s)
        s = jnp.where(ch, NEG, s)
        row = iota == i
        b1 = jnp.where(row, jnp.where(valid, rb1, 0.0), b1)
        b2 = jnp.where(row, jnp.where(valid, rb2, 0.0), b2)
        b3 = jnp.where(row, jnp.where(valid, rb3, 0.0), b3)
        b4 = jnp.where(row, jnp.where(valid, rb4, 0.0), b4)
        bs = jnp.where(row, jnp.where(valid, csc, 0.0), bs)
        bl = jnp.where(row, jnp.where(valid, clab, -1), bl)
        return (s, b1, b2, b3, b4, bs, bl)

    _, b1, b2, b3, b4, bs, bl = jax.lax.fori_loop(0, DET_PER_IMG, body, init)
    bx1_ref[...] = b1
    by1_ref[...] = b2
    bx2_ref[...] = b3
    by2_ref[...] = b4
    os_ref[...] = bs
    ol_ref[...] = bl


def _run_nms(topk_scores, labels, breg, anc):
    def p2(v):
        return jnp.pad(v, (0, PAD - TOPK)).reshape(8, 128)

    f32 = jax.ShapeDtypeStruct((8, 128), jnp.float32)
    i32 = jax.ShapeDtypeStruct((8, 128), jnp.int32)
    outs = pl.pallas_call(
        _nms_kernel,
        out_shape=[f32, f32, f32, f32, f32, i32],
    )(p2(topk_scores), p2(labels),
      p2(breg[:, 0]), p2(breg[:, 1]), p2(breg[:, 2]), p2(breg[:, 3]),
      p2(anc[:, 0]), p2(anc[:, 1]), p2(anc[:, 2]), p2(anc[:, 3]))
    bx1, by1, bx2, by2, osc, olb = [o.reshape(-1)[:DET_PER_IMG] for o in outs]
    out_boxes = jnp.stack([bx1, by1, bx2, by2], axis=1)
    return out_boxes, osc, olb


def kernel(cls_logits, bbox_regression, anchors):
    scores_flat = jax.nn.sigmoid(cls_logits).reshape(-1)
    scores_masked = jnp.where(scores_flat > SCORE_THRESH, scores_flat, 0.0)
    topk_scores, topk_idxs = jax.lax.top_k(scores_masked, TOPK)
    anchor_idxs = topk_idxs // C
    labels = topk_idxs % C
    breg = bbox_regression[anchor_idxs]
    anc = anchors[anchor_idxs]
    return _run_nms(topk_scores, labels, breg, anc)


# speedup vs baseline: 1.5024x; 1.5024x over previous
import math

import jax
import jax.numpy as jnp
from jax.experimental import pallas as pl

N = 20000
C = 91
SCORE_THRESH = 0.05
NMS_THRESH = 0.5
DET_PER_IMG = 300
TOPK = 1000
IMG_H = 800
IMG_W = 1066
BBOX_XFORM_CLIP = math.log(1000.0 / 16.0)
NEG = -1e9

PAD = 1024


def _nms_kernel(sc_ref, lb_ref, dx_ref, dy_ref, dw_ref, dh_ref,
                a1_ref, a2_ref, a3_ref, a4_ref,
                bx1_ref, by1_ref, bx2_ref, by2_ref, os_ref, ol_ref):
    score = sc_ref[...]
    label = lb_ref[...]
    a1 = a1_ref[...]
    a2 = a2_ref[...]
    a3 = a3_ref[...]
    a4 = a4_ref[...]

    widths = a3 - a1
    heights = a4 - a2
    ctr_x = a1 + 0.5 * widths
    ctr_y = a2 + 0.5 * heights
    dw = jnp.minimum(dw_ref[...], BBOX_XFORM_CLIP)
    dh = jnp.minimum(dh_ref[...], BBOX_XFORM_CLIP)
    pred_ctr_x = dx_ref[...] * widths + ctr_x
    pred_ctr_y = dy_ref[...] * heights + ctr_y
    pred_w = jnp.exp(dw) * widths
    pred_h = jnp.exp(dh) * heights
    x1 = jnp.clip(pred_ctr_x - 0.5 * pred_w, 0.0, float(IMG_W))
    y1 = jnp.clip(pred_ctr_y - 0.5 * pred_h, 0.0, float(IMG_H))
    x2 = jnp.clip(pred_ctr_x + 0.5 * pred_w, 0.0, float(IMG_W))
    y2 = jnp.clip(pred_ctr_y + 0.5 * pred_h, 0.0, float(IMG_H))

    maxbox = jnp.max(jnp.maximum(jnp.maximum(x1, y1), jnp.maximum(x2, y2)))
    off = label.astype(jnp.float32) * (maxbox + 1.0)
    o1 = x1 + off
    o2 = y1 + off
    o3 = x2 + off
    o4 = y2 + off
    area2 = (o3 - o1) * (o4 - o2)

    iota = (jax.lax.broadcasted_iota(jnp.int32, (8, 128), 0) * 128
            + jax.lax.broadcasted_iota(jnp.int32, (8, 128), 1))
    s0 = jnp.where(score > SCORE_THRESH, score, NEG)

    zf = jnp.zeros((8, 128), jnp.float32)
    init = (s0, zf, zf, zf, zf, zf, jnp.full((8, 128), -1, jnp.int32))

    def body(i, carry):
        s, b1, b2, b3, b4, bs, bl = carry
        m = jnp.max(s)
        eq = s == m
        idx = jnp.min(jnp.where(eq, iota, jnp.int32(1 << 30)))
        valid = m > NEG / 2.0
        ch = iota == idx
        rb1 = jnp.sum(jnp.where(ch, x1, 0.0))
        rb2 = jnp.sum(jnp.where(ch, y1, 0.0))
        rb3 = jnp.sum(jnp.where(ch, x2, 0.0))
        rb4 = jnp.sum(jnp.where(ch, y2, 0.0))
        coff = jnp.sum(jnp.where(ch, off, 0.0))
        csc = jnp.sum(jnp.where(ch, score, 0.0))
        clab = jnp.sum(jnp.where(ch, label, 0))
        cx1 = rb1 + coff
        cy1 = rb2 + coff
        cx2 = rb3 + coff
        cy2 = rb4 + coff
        area1 = (cx2 - cx1) * (cy2 - cy1)
        ltx = jnp.maximum(cx1, o1)
        lty = jnp.maximum(cy1, o2)
        rbx = jnp.minimum(cx2, o3)
        rby = jnp.minimum(cy2, o4)
        w = jnp.maximum(rbx - ltx, 0.0)
        h = jnp.maximum(rby - lty, 0.0)
        inter = w * h
        iou = inter / (area1 + area2 - inter + 1e-12)
        s = jnp.where(iou > NMS_THRESH, NEG, s)
        s = jnp.where(ch, NEG, s)
        row = iota == i
        b1 = jnp.where(row, jnp.where(valid, rb1, 0.0), b1)
        b2 = jnp.where(row, jnp.where(valid, rb2, 0.0), b2)
        b3 = jnp.where(row, jnp.where(valid, rb3, 0.0), b3)
        b4 = jnp.where(row, jnp.where(valid, rb4, 0.0), b4)
        bs = jnp.where(row, jnp.where(valid, csc, 0.0), bs)
        bl = jnp.where(row, jnp.where(valid, clab, -1), bl)
        return (s, b1, b2, b3, b4, bs, bl)

    _, b1, b2, b3, b4, bs, bl = jax.lax.fori_loop(0, DET_PER_IMG, body, init)
    bx1_ref[...] = b1
    by1_ref[...] = b2
    bx2_ref[...] = b3
    by2_ref[...] = b4
    os_ref[...] = bs
    ol_ref[...] = bl


def _run_nms(topk_scores, labels, breg, anc):
    def p2(v):
        return jnp.pad(v, (0, PAD - TOPK)).reshape(8, 128)

    f32 = jax.ShapeDtypeStruct((8, 128), jnp.float32)
    i32 = jax.ShapeDtypeStruct((8, 128), jnp.int32)
    outs = pl.pallas_call(
        _nms_kernel,
        out_shape=[f32, f32, f32, f32, f32, i32],
    )(p2(topk_scores), p2(labels),
      p2(breg[:, 0]), p2(breg[:, 1]), p2(breg[:, 2]), p2(breg[:, 3]),
      p2(anc[:, 0]), p2(anc[:, 1]), p2(anc[:, 2]), p2(anc[:, 3]))
    bx1, by1, bx2, by2, osc, olb = [o.reshape(-1)[:DET_PER_IMG] for o in outs]
    out_boxes = jnp.stack([bx1, by1, bx2, by2], axis=1)
    return out_boxes, osc, olb


def kernel(cls_logits, bbox_regression, anchors):
    scores_flat = jax.nn.sigmoid(cls_logits).reshape(-1)
    scores_masked = jnp.where(scores_flat > SCORE_THRESH, scores_flat, 0.0)
    rows = 16
    row_len = (N * C) // rows
    rs, ri = jax.lax.top_k(scores_masked.reshape(rows, row_len), TOPK)
    gi = ri + (jnp.arange(rows, dtype=ri.dtype) * row_len)[:, None]
    topk_scores, pos = jax.lax.top_k(rs.reshape(-1), TOPK)
    topk_idxs = gi.reshape(-1)[pos]
    anchor_idxs = topk_idxs // C
    labels = topk_idxs % C
    breg = bbox_regression[anchor_idxs]
    anc = anchors[anchor_idxs]
    return _run_nms(topk_scores, labels, breg, anc)


# speedup vs baseline: 1.5121x; 1.0065x over previous
import math

import jax
import jax.numpy as jnp
from jax.experimental import pallas as pl

N = 20000
C = 91
SCORE_THRESH = 0.05
NMS_THRESH = 0.5
DET_PER_IMG = 300
TOPK = 1000
IMG_H = 800
IMG_W = 1066
BBOX_XFORM_CLIP = math.log(1000.0 / 16.0)
NEG = -1e9

PAD = 1024


def _nms_kernel(sc_ref, lb_ref, dx_ref, dy_ref, dw_ref, dh_ref,
                a1_ref, a2_ref, a3_ref, a4_ref,
                bx1_ref, by1_ref, bx2_ref, by2_ref, os_ref, ol_ref):
    score = sc_ref[...]
    label = lb_ref[...]
    a1 = a1_ref[...]
    a2 = a2_ref[...]
    a3 = a3_ref[...]
    a4 = a4_ref[...]

    widths = a3 - a1
    heights = a4 - a2
    ctr_x = a1 + 0.5 * widths
    ctr_y = a2 + 0.5 * heights
    dw = jnp.minimum(dw_ref[...], BBOX_XFORM_CLIP)
    dh = jnp.minimum(dh_ref[...], BBOX_XFORM_CLIP)
    pred_ctr_x = dx_ref[...] * widths + ctr_x
    pred_ctr_y = dy_ref[...] * heights + ctr_y
    pred_w = jnp.exp(dw) * widths
    pred_h = jnp.exp(dh) * heights
    x1 = jnp.clip(pred_ctr_x - 0.5 * pred_w, 0.0, float(IMG_W))
    y1 = jnp.clip(pred_ctr_y - 0.5 * pred_h, 0.0, float(IMG_H))
    x2 = jnp.clip(pred_ctr_x + 0.5 * pred_w, 0.0, float(IMG_W))
    y2 = jnp.clip(pred_ctr_y + 0.5 * pred_h, 0.0, float(IMG_H))

    maxbox = jnp.max(jnp.maximum(jnp.maximum(x1, y1), jnp.maximum(x2, y2)))
    off = label.astype(jnp.float32) * (maxbox + 1.0)
    o1 = x1 + off
    o2 = y1 + off
    o3 = x2 + off
    o4 = y2 + off
    area2 = (o3 - o1) * (o4 - o2)

    iota = (jax.lax.broadcasted_iota(jnp.int32, (8, 128), 0) * 128
            + jax.lax.broadcasted_iota(jnp.int32, (8, 128), 1))
    s0 = jnp.where(score > SCORE_THRESH, score, NEG)

    zf = jnp.zeros((8, 128), jnp.float32)
    init = (s0, zf, zf, zf, zf, zf, jnp.full((8, 128), -1, jnp.int32))

    def body(i, carry):
        s, b1, b2, b3, b4, bs, bl = carry
        m = jnp.max(s)
        eq = s == m
        idx = jnp.min(jnp.where(eq, iota, jnp.int32(1 << 30)))
        valid = m > NEG / 2.0
        ch = iota == idx
        rb1 = jnp.sum(jnp.where(ch, x1, 0.0))
        rb2 = jnp.sum(jnp.where(ch, y1, 0.0))
        rb3 = jnp.sum(jnp.where(ch, x2, 0.0))
        rb4 = jnp.sum(jnp.where(ch, y2, 0.0))
        coff = jnp.sum(jnp.where(ch, off, 0.0))
        csc = jnp.sum(jnp.where(ch, score, 0.0))
        clab = jnp.sum(jnp.where(ch, label, 0))
        cx1 = rb1 + coff
        cy1 = rb2 + coff
        cx2 = rb3 + coff
        cy2 = rb4 + coff
        area1 = (cx2 - cx1) * (cy2 - cy1)
        ltx = jnp.maximum(cx1, o1)
        lty = jnp.maximum(cy1, o2)
        rbx = jnp.minimum(cx2, o3)
        rby = jnp.minimum(cy2, o4)
        w = jnp.maximum(rbx - ltx, 0.0)
        h = jnp.maximum(rby - lty, 0.0)
        inter = w * h
        iou = inter / (area1 + area2 - inter + 1e-12)
        s = jnp.where(iou > NMS_THRESH, NEG, s)
        s = jnp.where(ch, NEG, s)
        row = iota == i
        b1 = jnp.where(row, jnp.where(valid, rb1, 0.0), b1)
        b2 = jnp.where(row, jnp.where(valid, rb2, 0.0), b2)
        b3 = jnp.where(row, jnp.where(valid, rb3, 0.0), b3)
        b4 = jnp.where(row, jnp.where(valid, rb4, 0.0), b4)
        bs = jnp.where(row, jnp.where(valid, csc, 0.0), bs)
        bl = jnp.where(row, jnp.where(valid, clab, -1), bl)
        return (s, b1, b2, b3, b4, bs, bl)

    _, b1, b2, b3, b4, bs, bl = jax.lax.fori_loop(0, DET_PER_IMG, body, init)
    bx1_ref[...] = b1
    by1_ref[...] = b2
    bx2_ref[...] = b3
    by2_ref[...] = b4
    os_ref[...] = bs
    ol_ref[...] = bl


def _run_nms(topk_scores, labels, breg, anc):
    def p2(v):
        return jnp.pad(v, (0, PAD - TOPK)).reshape(8, 128)

    f32 = jax.ShapeDtypeStruct((8, 128), jnp.float32)
    i32 = jax.ShapeDtypeStruct((8, 128), jnp.int32)
    outs = pl.pallas_call(
        _nms_kernel,
        out_shape=[f32, f32, f32, f32, f32, i32],
    )(p2(topk_scores), p2(labels),
      p2(breg[:, 0]), p2(breg[:, 1]), p2(breg[:, 2]), p2(breg[:, 3]),
      p2(anc[:, 0]), p2(anc[:, 1]), p2(anc[:, 2]), p2(anc[:, 3]))
    bx1, by1, bx2, by2, osc, olb = [o.reshape(-1)[:DET_PER_IMG] for o in outs]
    out_boxes = jnp.stack([bx1, by1, bx2, by2], axis=1)
    return out_boxes, osc, olb


def kernel(cls_logits, bbox_regression, anchors):
    scores_flat = jax.nn.sigmoid(cls_logits).reshape(-1)
    scores_masked = jnp.where(scores_flat > SCORE_THRESH, scores_flat, 0.0)
    rows = 40
    row_len = (N * C) // rows
    rs, ri = jax.lax.top_k(scores_masked.reshape(rows, row_len), TOPK)
    gi = ri + (jnp.arange(rows, dtype=ri.dtype) * row_len)[:, None]
    topk_scores, pos = jax.lax.top_k(rs.reshape(-1), TOPK)
    topk_idxs = gi.reshape(-1)[pos]
    anchor_idxs = topk_idxs // C
    labels = topk_idxs % C
    breg = bbox_regression[anchor_idxs]
    anc = anchors[anchor_idxs]
    return _run_nms(topk_scores, labels, breg, anc)
